# baseline (speedup 1.0000x reference)
import functools

import jax
import jax.numpy as jnp
from jax import lax
from jax.experimental import pallas as pl
from jax.experimental.pallas import tpu as pltpu
from jax.experimental.pallas import tpu_sc as plsc

_C = 128
_DEPTH = 8


@functools.lru_cache(maxsize=None)
def _build(n, v, d):
    info = plsc.get_sparse_core_info()
    nw = info.num_cores * info.num_subcores
    per_w = n // nw
    nch = per_w // _C
    ng = nch // _DEPTH
    assert per_w % _C == 0 and nch % _DEPTH == 0

    mesh = plsc.VectorSubcoreMesh(core_axis_name="c", subcore_axis_name="s")

    def body(table_hbm, idx_hbm, out_hbm, idx_v, rows_v, *sems):
        gsem = sems[:_DEPTH]
        osem = sems[_DEPTH:]
        wid = lax.axis_index("s") * info.num_cores + lax.axis_index("c")
        pltpu.sync_copy(idx_hbm.at[wid], idx_v)

        def group(j, carry):
            gathers = []
            for b in range(_DEPTH):
                i = j * _DEPTH + b
                cp = pltpu.async_copy(
                    table_hbm.at[idx_v.at[i]], rows_v.at[b], gsem[b])
                gathers.append((cp, i))
            outs = []
            for b, (cp, i) in enumerate(gathers):
                cp.wait()
                outs.append(pltpu.async_copy(
                    rows_v.at[b], out_hbm.at[wid, i], osem[b]))
            for cp in outs:
                cp.wait()
            return carry

        lax.fori_loop(0, ng, group, 0)

    grid_kernel = pl.kernel(
        body,
        out_type=jax.ShapeDtypeStruct((nw, nch, _C, d), jnp.float32),
        mesh=mesh,
        scratch_types=(
            [pltpu.VMEM((nch, _C), jnp.int32),
             pltpu.VMEM((_DEPTH, _C, d), jnp.float32)]
            + [pltpu.SemaphoreType.DMA] * (2 * _DEPTH)
        ),
        compiler_params=pltpu.CompilerParams(use_tc_tiling_on_sc=False),
    )
    return grid_kernel, nw, nch


def kernel(input_ids, table):
    s, b = input_ids.shape
    v, d = table.shape
    n = s * b
    gather, nw, nch = _build(n, v, d)
    idx = input_ids.reshape(nw, nch, _C)
    out = gather(table, idx)
    return out.reshape(s, b, d)
